# Initial kernel scaffold; baseline (speedup 1.0000x reference)
import jax
import jax.numpy as jnp
from jax.experimental import pallas as pl


def kernel(x_len, index, pe):
    raise NotImplementedError("write your pallas kernel here")


# speedup vs baseline: 1.3840x; 1.3840x over previous
import functools

import jax
import jax.numpy as jnp
from jax import lax
from jax.experimental import pallas as pl
from jax.experimental.pallas import tpu as pltpu
from jax.experimental.pallas import tpu_sc as plsc

D_MODEL = 1024

_info = plsc.get_sparse_core_info()
_NC, _NS = _info.num_cores, _info.num_subcores
_NW = _NC * _NS


def _make_gather(n_rows: int, d: int):
    rows_per_w = n_rows // _NW
    chunk = 64
    n_chunks = rows_per_w // chunk
    mesh = plsc.VectorSubcoreMesh(core_axis_name="c", subcore_axis_name="s")

    @functools.partial(
        pl.kernel,
        mesh=mesh,
        out_type=jax.ShapeDtypeStruct((n_rows, d), jnp.float32),
        scratch_types=[
            pltpu.VMEM((chunk,), jnp.int32),
            pltpu.VMEM((chunk, d), jnp.float32),
            pltpu.SemaphoreType.DMA,
        ],
    )
    def gather_kernel(table_hbm, idx_hbm, out_hbm, idx_v, rows_v, sem):
        wid = lax.axis_index("s") * _NC + lax.axis_index("c")
        base = wid * rows_per_w

        @pl.loop(0, n_chunks)
        def _(c):
            off = base + c * chunk
            pltpu.sync_copy(idx_hbm.at[pl.ds(off, chunk)], idx_v)
            pltpu.async_copy(table_hbm.at[idx_v], rows_v, sem).wait()
            pltpu.sync_copy(rows_v, out_hbm.at[pl.ds(off, chunk)])

    return gather_kernel


def kernel(x_len, index, pe):
    if index is None:
        return pe[:, :x_len, :]
    b, s, _ = index.shape
    table = pe[0]
    idx_flat = index.reshape(b * s).astype(jnp.int32)
    out = _make_gather(b * s, table.shape[1])(table, idx_flat)
    return out.reshape(b, s, table.shape[1])


# speedup vs baseline: 1.4117x; 1.0200x over previous
import functools

import jax
import jax.numpy as jnp
from jax import lax
from jax.experimental import pallas as pl
from jax.experimental.pallas import tpu as pltpu
from jax.experimental.pallas import tpu_sc as plsc

D_MODEL = 1024

_info = plsc.get_sparse_core_info()
_NC, _NS = _info.num_cores, _info.num_subcores
_NW = _NC * _NS


def _make_gather(n_rows: int, d: int):
    rows_per_w = n_rows // _NW
    chunk = 32
    n_chunks = rows_per_w // chunk
    mesh = plsc.VectorSubcoreMesh(core_axis_name="c", subcore_axis_name="s")

    @functools.partial(
        pl.kernel,
        mesh=mesh,
        out_type=jax.ShapeDtypeStruct((n_rows, d), jnp.float32),
        scratch_types=[
            pltpu.VMEM((n_chunks, chunk), jnp.int32),
            pltpu.VMEM((chunk, d), jnp.float32),
            pltpu.VMEM((chunk, d), jnp.float32),
            pltpu.SemaphoreType.DMA,
            pltpu.SemaphoreType.DMA,
            pltpu.SemaphoreType.DMA,
            pltpu.SemaphoreType.DMA,
        ],
    )
    def gather_kernel(table_hbm, idx_hbm, out_hbm, idx_v, rows0, rows1,
                      g0, g1, s0, s1):
        wid = lax.axis_index("s") * _NC + lax.axis_index("c")
        base = wid * rows_per_w
        rows = (rows0, rows1)
        gsem = (g0, g1)
        ssem = (s0, s1)

        pltpu.sync_copy(idx_hbm.at[wid], idx_v)

        def gather(c):
            return pltpu.async_copy(
                table_hbm.at[idx_v.at[c]], rows[c % 2], gsem[c % 2])

        def store(c):
            return pltpu.async_copy(
                rows[c % 2], out_hbm.at[pl.ds(base + c * chunk, chunk)],
                ssem[c % 2])

        stores = [None, None]
        gather(0).wait()
        for c in range(n_chunks):
            if c + 1 < n_chunks:
                if stores[(c + 1) % 2] is not None:
                    stores[(c + 1) % 2].wait()
                nxt = gather(c + 1)
            stores[c % 2] = store(c)
            if c + 1 < n_chunks:
                nxt.wait()
        for s in stores:
            if s is not None:
                s.wait()

    return gather_kernel


def kernel(x_len, index, pe):
    if index is None:
        return pe[:, :x_len, :]
    b, s, _ = index.shape
    table = pe[0]
    n = b * s
    chunk = 32
    idx_3d = index.reshape(_NW, n // (_NW * chunk), chunk).astype(jnp.int32)
    out = _make_gather(n, table.shape[1])(table, idx_3d)
    return out.reshape(b, s, table.shape[1])


# speedup vs baseline: 1.4396x; 1.0198x over previous
import functools

import jax
import jax.numpy as jnp
from jax import lax
from jax.experimental import pallas as pl
from jax.experimental.pallas import tpu as pltpu
from jax.experimental.pallas import tpu_sc as plsc

D_MODEL = 1024

_info = plsc.get_sparse_core_info()
_NC, _NS = _info.num_cores, _info.num_subcores
_NW = _NC * _NS


def _make_gather(n_rows: int, d: int):
    rows_per_w = n_rows // _NW
    chunk = 16
    n_chunks = rows_per_w // chunk
    mesh = plsc.VectorSubcoreMesh(core_axis_name="c", subcore_axis_name="s")

    @functools.partial(
        pl.kernel,
        mesh=mesh,
        out_type=jax.ShapeDtypeStruct((n_rows, d), jnp.float32),
        scratch_types=[
            pltpu.VMEM((n_chunks, chunk), jnp.int32),
        ] + [pltpu.VMEM((chunk, d), jnp.float32)] * 6
          + [pltpu.SemaphoreType.DMA] * 12,
    )
    def gather_kernel(table_hbm, idx_hbm, out_hbm, idx_v, *bufs):
        rows = bufs[:6]
        gsem = bufs[6:12]
        ssem = bufs[12:18]
        wid = lax.axis_index("s") * _NC + lax.axis_index("c")
        base = wid * rows_per_w

        pltpu.sync_copy(idx_hbm.at[wid], idx_v)

        NB = 6

        def gather(c):
            return pltpu.async_copy(
                table_hbm.at[idx_v.at[c]], rows[c % NB], gsem[c % NB])

        def store(c):
            return pltpu.async_copy(
                rows[c % NB], out_hbm.at[pl.ds(base + c * chunk, chunk)],
                ssem[c % NB])

        LOOKAHEAD = 3
        pend_g = {c: gather(c) for c in range(LOOKAHEAD)}
        pend_s = {}
        for c in range(n_chunks):
            nxt = c + LOOKAHEAD
            if nxt < n_chunks:
                prev = nxt - NB
                if prev in pend_s:
                    pend_s.pop(prev).wait()
                pend_g[nxt] = gather(nxt)
            pend_g.pop(c).wait()
            pend_s[c] = store(c)
        for c in sorted(pend_s):
            pend_s[c].wait()

    return gather_kernel


def kernel(x_len, index, pe):
    if index is None:
        return pe[:, :x_len, :]
    b, s, _ = index.shape
    table = pe[0]
    n = b * s
    chunk = 16
    idx_3d = index.reshape(_NW, n // (_NW * chunk), chunk).astype(jnp.int32)
    out = _make_gather(n, table.shape[1])(table, idx_3d)
    return out.reshape(b, s, table.shape[1])


# speedup vs baseline: 1.4401x; 1.0003x over previous
import functools

import jax
import jax.numpy as jnp
from jax import lax
from jax.experimental import pallas as pl
from jax.experimental.pallas import tpu as pltpu
from jax.experimental.pallas import tpu_sc as plsc

D_MODEL = 1024

_info = plsc.get_sparse_core_info()
_NC, _NS = _info.num_cores, _info.num_subcores
_NW = _NC * _NS


def _make_gather(n_batch: int, n_seq: int, d: int):
    n_rows = n_batch * n_seq
    rows_per_w = n_rows // _NW
    w_per_batch = _NW // n_batch
    chunk = 16
    n_chunks = rows_per_w // chunk
    mesh = plsc.VectorSubcoreMesh(core_axis_name="c", subcore_axis_name="s")

    @functools.partial(
        pl.kernel,
        mesh=mesh,
        out_type=jax.ShapeDtypeStruct((n_batch, n_seq, d), jnp.float32),
        scratch_types=[
            pltpu.VMEM((n_chunks, chunk), jnp.int32),
        ] + [pltpu.VMEM((chunk, d), jnp.float32)] * 6
          + [pltpu.SemaphoreType.DMA] * 12,
    )
    def gather_kernel(table_hbm, idx_hbm, out_hbm, idx_v, *bufs):
        rows = bufs[:6]
        gsem = bufs[6:12]
        ssem = bufs[12:18]
        wid = lax.axis_index("s") * _NC + lax.axis_index("c")
        batch = wid // w_per_batch
        base = (wid % w_per_batch) * rows_per_w

        pltpu.sync_copy(idx_hbm.at[wid], idx_v)

        NB = 6

        def gather(c):
            return pltpu.async_copy(
                table_hbm.at[idx_v.at[c]], rows[c % NB], gsem[c % NB])

        def store(c):
            return pltpu.async_copy(
                rows[c % NB],
                out_hbm.at[batch, pl.ds(base + c * chunk, chunk)],
                ssem[c % NB])

        LOOKAHEAD = 3
        pend_g = {c: gather(c) for c in range(LOOKAHEAD)}
        pend_s = {}
        for c in range(n_chunks):
            nxt = c + LOOKAHEAD
            if nxt < n_chunks:
                prev = nxt - NB
                if prev in pend_s:
                    pend_s.pop(prev).wait()
                pend_g[nxt] = gather(nxt)
            pend_g.pop(c).wait()
            pend_s[c] = store(c)
        for c in sorted(pend_s):
            pend_s[c].wait()

    return gather_kernel


def kernel(x_len, index, pe):
    if index is None:
        return pe[:, :x_len, :]
    b, s, _ = index.shape
    table = pe[0]
    n = b * s
    chunk = 16
    idx_3d = index.reshape(_NW, n // (_NW * chunk), chunk).astype(jnp.int32)
    return _make_gather(b, s, table.shape[1])(table, idx_3d)


# speedup vs baseline: 1.5569x; 1.0811x over previous
import functools

import jax
import jax.numpy as jnp
from jax import lax
from jax.experimental import pallas as pl
from jax.experimental.pallas import tpu as pltpu
from jax.experimental.pallas import tpu_sc as plsc

D_MODEL = 1024

_info = plsc.get_sparse_core_info()
_NC, _NS = _info.num_cores, _info.num_subcores
_NW = _NC * _NS


def _make_gather(n_batch: int, n_seq: int, d: int):
    n_rows = n_batch * n_seq
    rows_per_w = n_rows // _NW
    w_per_batch = _NW // n_batch
    chunk = 16
    n_chunks = rows_per_w // chunk
    mesh = plsc.VectorSubcoreMesh(core_axis_name="c", subcore_axis_name="s")

    @functools.partial(
        pl.kernel,
        mesh=mesh,
        out_type=jax.ShapeDtypeStruct((n_batch, n_seq, d), jnp.float32),
        scratch_types=[
            pltpu.VMEM((n_chunks, chunk), jnp.int32),
        ] + [pltpu.VMEM((chunk, d), jnp.float32)] * 6
          + [pltpu.SemaphoreType.DMA] * 12,
    )
    def gather_kernel(table_hbm, idx_hbm, out_hbm, idx_v, *bufs):
        rows = bufs[:6]
        gsem = bufs[6:12]
        ssem = bufs[12:18]
        wid = lax.axis_index("s") * _NC + lax.axis_index("c")
        batch = wid // w_per_batch
        base = (wid % w_per_batch) * rows_per_w

        pltpu.sync_copy(idx_hbm.at[wid], idx_v)

        NB = 6

        def gather(c):
            return pltpu.async_copy(
                table_hbm.at[0].at[idx_v.at[c]], rows[c % NB],
                gsem[c % NB])

        def store(c):
            return pltpu.async_copy(
                rows[c % NB],
                out_hbm.at[batch, pl.ds(base + c * chunk, chunk)],
                ssem[c % NB])

        LOOKAHEAD = 3
        pend_g = {c: gather(c) for c in range(LOOKAHEAD)}
        pend_s = {}
        for c in range(n_chunks):
            nxt = c + LOOKAHEAD
            if nxt < n_chunks:
                prev = nxt - NB
                if prev in pend_s:
                    pend_s.pop(prev).wait()
                pend_g[nxt] = gather(nxt)
            pend_g.pop(c).wait()
            pend_s[c] = store(c)
        for c in sorted(pend_s):
            pend_s[c].wait()

    return gather_kernel


def kernel(x_len, index, pe):
    if index is None:
        return pe[:, :x_len, :]
    b, s, _ = index.shape
    n = b * s
    chunk = 16
    idx_3d = index.reshape(_NW, n // (_NW * chunk), chunk).astype(jnp.int32)
    return _make_gather(b, s, pe.shape[2])(pe, idx_3d)
